# baseline (device time: 11634 ns/iter reference)
import jax
import jax.numpy as jnp
from jax import lax
from jax.experimental import pallas as pl
from jax.experimental.pallas import tpu as pltpu

N_DEV = 4
S = 2

RAW_A = 0
RAW_B = 1
DIR_A = 2
DIR_B = 3
PART_A = 4
PART_B = 5

S_RAW_A = 0
S_RAW_B = S
S_DIR_A = 2 * S
S_DIR_B = 2 * S + 1
S_PART_A = 2 * S + 2
S_PART_B = 3 * S + 2
N_SEM = 4 * S + 2

V_OPP, V_LEFT, V_RIGHT, V_MINE = 0, 1, 2, 3


def kernel(x):
    _, m, n_total = x.shape
    n_per = n_total // N_DEV
    h = m // 2
    hp = h // S

    def body(x_hbm, out_hbm, xv, recv_buf, stage_buf, out_v, send_sems,
             recv_sems, cp_sems, out_cp_sems):
        my = lax.axis_index("i")
        left = lax.rem(my + N_DEV - 1, N_DEV)
        right = lax.rem(my + 1, N_DEV)

        cps = []
        for slot, dev in ((V_OPP, lax.rem(my + 2, N_DEV)), (V_LEFT, left),
                          (V_RIGHT, right), (V_MINE, my)):
            cp = pltpu.make_async_copy(
                x_hbm.at[0, :, pl.ds(dev * n_per, n_per)],
                xv.at[slot],
                cp_sems.at[slot],
            )
            cp.start()
            cps.append(cp)

        barrier_sem = pltpu.get_barrier_semaphore()
        for nbr in [left, right]:
            pl.semaphore_signal(
                barrier_sem, inc=1,
                device_id=(nbr,), device_id_type=pl.DeviceIdType.MESH,
            )
        pl.semaphore_wait(barrier_sem, 2)

        def copy(src, slot, rows, sem_idx, dst_dev):
            return pltpu.make_async_remote_copy(
                src_ref=src,
                dst_ref=recv_buf.at[slot, rows, :],
                send_sem=send_sems.at[sem_idx],
                recv_sem=recv_sems.at[sem_idx],
                device_id=(dst_dev,),
                device_id_type=pl.DeviceIdType.MESH,
            )

        cps[V_OPP].wait()
        raws = []
        for s in range(S):
            ra = copy(xv.at[V_OPP, pl.ds(s * hp, hp), :],
                      RAW_A, pl.ds(s * hp, hp), S_RAW_A + s, left)
            rb = copy(xv.at[V_OPP, pl.ds(h + s * hp, hp), :],
                      RAW_B, pl.ds(s * hp, hp), S_RAW_B + s, right)
            ra.start()
            rb.start()
            raws.append((ra, rb))

        cps[V_RIGHT].wait()
        dir_a = copy(xv.at[V_RIGHT, pl.ds(0, h), :], DIR_A, pl.ds(0, h),
                     S_DIR_A, right)
        dir_a.start()
        cps[V_LEFT].wait()
        dir_b = copy(xv.at[V_LEFT, pl.ds(h, h), :], DIR_B, pl.ds(0, h),
                     S_DIR_B, left)
        dir_b.start()

        parts = []
        for s in range(S):
            ra, rb = raws[s]
            rows = pl.ds(s * hp, hp)
            ra.wait_recv()
            stage_buf[0, rows, :] = (
                xv[V_LEFT, rows, :] + recv_buf[RAW_A, rows, :]
            )
            pa = copy(stage_buf.at[0, rows, :], PART_A, rows,
                      S_PART_A + s, left)
            pa.start()
            rb.wait_recv()
            stage_buf[1, rows, :] = (
                xv[V_RIGHT, pl.ds(h + s * hp, hp), :] + recv_buf[RAW_B, rows, :]
            )
            pb = copy(stage_buf.at[1, rows, :], PART_B, rows,
                      S_PART_B + s, right)
            pb.start()
            parts.append((pa, pb))

        cps[V_MINE].wait()
        dir_a.wait_recv()
        acc_t = xv[V_MINE, pl.ds(0, h), :] + recv_buf[DIR_A, :, :]
        dir_b.wait_recv()
        acc_b = xv[V_MINE, pl.ds(h, h), :] + recv_buf[DIR_B, :, :]
        for pa, _ in parts:
            pa.wait_recv()
        out_v[pl.ds(0, h), :] = acc_t + recv_buf[PART_A, :, :]
        cp_t = pltpu.make_async_copy(
            out_v.at[pl.ds(0, h), :], out_hbm.at[pl.ds(0, h), :],
            out_cp_sems.at[0])
        cp_t.start()
        for _, pb in parts:
            pb.wait_recv()
        out_v[pl.ds(h, h), :] = acc_b + recv_buf[PART_B, :, :]
        cp_b = pltpu.make_async_copy(
            out_v.at[pl.ds(h, h), :], out_hbm.at[pl.ds(h, h), :],
            out_cp_sems.at[1])
        cp_b.start()
        cp_t.wait()
        cp_b.wait()

        for ra, rb in raws:
            ra.wait_send()
            rb.wait_send()
        dir_a.wait_send()
        dir_b.wait_send()
        for pa, pb in parts:
            pa.wait_send()
            pb.wait_send()

    return pl.pallas_call(
        body,
        out_shape=jax.ShapeDtypeStruct((m, n_per), x.dtype),
        in_specs=[pl.BlockSpec(memory_space=pltpu.MemorySpace.HBM)],
        out_specs=pl.BlockSpec(memory_space=pltpu.MemorySpace.HBM),
        scratch_shapes=[
            pltpu.VMEM((4, m, n_per), x.dtype),
            pltpu.VMEM((6, h, n_per), x.dtype),
            pltpu.VMEM((2, h, n_per), x.dtype),
            pltpu.VMEM((m, n_per), x.dtype),
            pltpu.SemaphoreType.DMA((N_SEM,)),
            pltpu.SemaphoreType.DMA((N_SEM,)),
            pltpu.SemaphoreType.DMA((4,)),
            pltpu.SemaphoreType.DMA((2,)),
        ],
        compiler_params=pltpu.CompilerParams(collective_id=0),
    )(x)


# device time: 11143 ns/iter; 1.0441x vs baseline; 1.0441x over previous
import jax
import jax.numpy as jnp
from jax import lax
from jax.experimental import pallas as pl
from jax.experimental.pallas import tpu as pltpu

N_DEV = 4
S = 2

RAW_A = 0
RAW_B = 1
DIR_A = 2
DIR_B = 3
PART_A = 4
PART_B = 5

S_RAW_A = 0
S_RAW_B = S
S_DIR_A = 2 * S
S_DIR_B = 2 * S + 1
S_PART_A = 2 * S + 2
S_PART_B = 3 * S + 2
N_SEM = 4 * S + 2


def kernel(x):
    _, m, n_total = x.shape
    n_per = n_total // N_DEV
    h = m // 2
    hp = h // S

    def body(x_ref, out_ref, recv_buf, stage_buf, send_sems, recv_sems):
        my = lax.axis_index("i")
        left = lax.rem(my + N_DEV - 1, N_DEV)
        right = lax.rem(my + 1, N_DEV)
        c_left = left * n_per
        c_right = right * n_per
        c_opp = lax.rem(my + 2, N_DEV) * n_per
        c_mine = my * n_per

        barrier_sem = pltpu.get_barrier_semaphore()
        for nbr in [left, right]:
            pl.semaphore_signal(
                barrier_sem, inc=1,
                device_id=(nbr,), device_id_type=pl.DeviceIdType.MESH,
            )
        pl.semaphore_wait(barrier_sem, 2)

        def copy(src, slot, rows, sem_idx, dst_dev):
            return pltpu.make_async_remote_copy(
                src_ref=src,
                dst_ref=recv_buf.at[slot, rows, :],
                send_sem=send_sems.at[sem_idx],
                recv_sem=recv_sems.at[sem_idx],
                device_id=(dst_dev,),
                device_id_type=pl.DeviceIdType.MESH,
            )

        raws = []
        for s in range(S):
            ra = copy(x_ref.at[0, pl.ds(s * hp, hp), pl.ds(c_opp, n_per)],
                      RAW_A, pl.ds(s * hp, hp), S_RAW_A + s, left)
            rb = copy(x_ref.at[0, pl.ds(h + s * hp, hp), pl.ds(c_opp, n_per)],
                      RAW_B, pl.ds(s * hp, hp), S_RAW_B + s, right)
            ra.start()
            rb.start()
            raws.append((ra, rb))

        dir_a = copy(x_ref.at[0, pl.ds(0, h), pl.ds(c_right, n_per)],
                     DIR_A, pl.ds(0, h), S_DIR_A, right)
        dir_b = copy(x_ref.at[0, pl.ds(h, h), pl.ds(c_left, n_per)],
                     DIR_B, pl.ds(0, h), S_DIR_B, left)
        dir_a.start()
        dir_b.start()

        parts = []
        for s in range(S):
            ra, rb = raws[s]
            rows = pl.ds(s * hp, hp)
            ra.wait_recv()
            stage_buf[0, rows, :] = (
                x_ref[0, pl.ds(s * hp, hp), pl.ds(c_left, n_per)]
                + recv_buf[RAW_A, rows, :]
            )
            pa = copy(stage_buf.at[0, rows, :], PART_A, rows,
                      S_PART_A + s, left)
            pa.start()
            rb.wait_recv()
            stage_buf[1, rows, :] = (
                x_ref[0, pl.ds(h + s * hp, hp), pl.ds(c_right, n_per)]
                + recv_buf[RAW_B, rows, :]
            )
            pb = copy(stage_buf.at[1, rows, :], PART_B, rows,
                      S_PART_B + s, right)
            pb.start()
            parts.append((pa, pb))

        dir_a.wait_recv()
        acc_t = x_ref[0, pl.ds(0, h), pl.ds(c_mine, n_per)] + recv_buf[DIR_A]
        dir_b.wait_recv()
        acc_b = x_ref[0, pl.ds(h, h), pl.ds(c_mine, n_per)] + recv_buf[DIR_B]
        for pa, _ in parts:
            pa.wait_recv()
        out_ref[pl.ds(0, h), :] = acc_t + recv_buf[PART_A]
        for _, pb in parts:
            pb.wait_recv()
        out_ref[pl.ds(h, h), :] = acc_b + recv_buf[PART_B]

        for ra, rb in raws:
            ra.wait_send()
            rb.wait_send()
        dir_a.wait_send()
        dir_b.wait_send()
        for pa, pb in parts:
            pa.wait_send()
            pb.wait_send()

    return pl.pallas_call(
        body,
        out_shape=jax.ShapeDtypeStruct((m, n_per), x.dtype),
        in_specs=[pl.BlockSpec(memory_space=pltpu.VMEM)],
        out_specs=pl.BlockSpec(memory_space=pltpu.VMEM),
        scratch_shapes=[
            pltpu.VMEM((6, h, n_per), x.dtype),
            pltpu.VMEM((2, h, n_per), x.dtype),
            pltpu.SemaphoreType.DMA((N_SEM,)),
            pltpu.SemaphoreType.DMA((N_SEM,)),
        ],
        compiler_params=pltpu.CompilerParams(collective_id=0),
    )(x)
